# Initial kernel scaffold; baseline (speedup 1.0000x reference)
import jax
import jax.numpy as jnp
from jax.experimental import pallas as pl


def kernel(x, edge_index, W1, b1, g1, be1, W2, b2, g2, be2):
    raise NotImplementedError("write your pallas kernel here")


# speedup vs baseline: 8.5085x; 8.5085x over previous
import functools

import jax
import jax.numpy as jnp
from jax import lax
from jax.experimental import pallas as pl
from jax.experimental.pallas import tpu as pltpu
from jax.experimental.pallas import tpu_sc as plsc

N = 10000
NP = 10240
E = 320000
EP = 327680
NCHUNK = EP // 128
CPW_AGG = NCHUNK // 16
CPW_DEG = NCHUNK // 32
RPS = NP // 16
IN_DIM = 128
HID = 256
EPS = 1e-5

_MESH = plsc.VectorSubcoreMesh(core_axis_name="c", subcore_axis_name="s",
                               num_cores=2, num_subcores=16)


@functools.partial(
    pl.kernel,
    out_type=jax.ShapeDtypeStruct((2 * NP, 128), jnp.float32),
    mesh=_MESH,
    scratch_types=[
        pltpu.VMEM((CPW_DEG, 128), jnp.int32),
        pltpu.VMEM((128, 128), jnp.float32),
        pltpu.VMEM_SHARED((NP, 128), jnp.float32),
    ],
)
def _deg_kernel(idst_hbm, out_hbm, idx_v, ones_v, acc):
    c = lax.axis_index("c")
    s = lax.axis_index("s")

    @pl.loop(0, 128)
    def _(i):
        for j in range(8):
            ones_v[i, pl.ds(j * 16, 16)] = jnp.zeros((16,), jnp.float32)

    for k in range(RPS // 128):
        pltpu.sync_copy(ones_v, acc.at[pl.ds(s * RPS + k * 128, 128)])

    w = s * 2 + c
    pltpu.sync_copy(idst_hbm.at[pl.ds(w * CPW_DEG, CPW_DEG)], idx_v)
    plsc.subcore_barrier()

    @pl.loop(0, 128)
    def _(i):
        for j in range(8):
            ones_v[i, pl.ds(j * 16, 16)] = jnp.full((16,), 1.0, jnp.float32)

    @pl.loop(0, CPW_DEG)
    def _(j):
        pltpu.sync_copy(ones_v, acc.at[idx_v.at[j]], add=True)

    plsc.subcore_barrier()
    pltpu.sync_copy(acc.at[pl.ds(s * RPS, RPS)],
                    out_hbm.at[pl.ds(c * NP + s * RPS, RPS)])


IB = 16
NB = CPW_AGG // IB


@functools.partial(
    pl.kernel,
    out_type=jax.ShapeDtypeStruct((2 * NP, 128), jnp.float32),
    mesh=_MESH,
    scratch_types=[
        pltpu.VMEM((IB, 128), jnp.int32),
        pltpu.VMEM((IB, 128), jnp.int32),
        pltpu.VMEM((128, 128), jnp.float32),
        pltpu.VMEM((128, 128), jnp.float32),
        pltpu.VMEM_SHARED((NP, 128), jnp.float32),
        pltpu.SemaphoreType.DMA,
        pltpu.SemaphoreType.DMA,
    ],
)
def _agg_kernel(table_hbm, isrc_hbm, idst_hbm, out_hbm,
                isrc_v, idst_v, rows_a, rows_b, acc, sem_a, sem_b):
    c = lax.axis_index("c")
    s = lax.axis_index("s")

    @pl.loop(0, 128)
    def _(i):
        for j in range(8):
            rows_a[i, pl.ds(j * 16, 16)] = jnp.zeros((16,), jnp.float32)

    for k in range(RPS // 128):
        pltpu.sync_copy(rows_a, acc.at[pl.ds(s * RPS + k * 128, 128)])

    plsc.subcore_barrier()

    @pl.loop(0, NB)
    def _(t):
        base = s * CPW_AGG + t * IB
        pltpu.sync_copy(isrc_hbm.at[pl.ds(c * NCHUNK + base, IB)], isrc_v)
        pltpu.sync_copy(idst_hbm.at[pl.ds(base, IB)], idst_v)

        @pl.loop(0, IB, step=2)
        def _(j):
            cp_a = pltpu.async_copy(table_hbm.at[isrc_v.at[j]], rows_a, sem_a)
            cp_b = pltpu.async_copy(table_hbm.at[isrc_v.at[j + 1]], rows_b,
                                    sem_b)
            cp_a.wait()
            pltpu.sync_copy(rows_a, acc.at[idst_v.at[j]], add=True)
            cp_b.wait()
            pltpu.sync_copy(rows_b, acc.at[idst_v.at[j + 1]], add=True)

    plsc.subcore_barrier()
    pltpu.sync_copy(acc.at[pl.ds(s * RPS, RPS)],
                    out_hbm.at[pl.ds(c * NP + s * RPS, RPS)])


def _dinv_from_deg(deg_parts):
    deg = jnp.max(deg_parts[0:NP] + deg_parts[NP:2 * NP], axis=1) + 1.0
    return lax.rsqrt(deg)[:N]


def _write_table(table_ref, hp):
    table_ref[0:N, :] = hp[:, :128]
    table_ref[N:NP, :] = jnp.zeros((NP - N, 128), jnp.float32)
    table_ref[NP:NP + N, :] = hp[:, 128:]
    table_ref[NP + N:2 * NP, :] = jnp.zeros((NP - N, 128), jnp.float32)


def _mm_body(x_ref, w_ref, o_ref):
    o_ref[...] = jnp.dot(x_ref[...], w_ref[...],
                         preferred_element_type=jnp.float32)


def _prep_body(h_ref, deg_ref, table_ref):
    dinv = _dinv_from_deg(deg_ref[...])
    _write_table(table_ref, h_ref[...] * dinv[:, None])


def _conv_body(agg_ref, table_ref, deg_ref, b_ref, conv_ref):
    dinv = _dinv_from_deg(deg_ref[...])
    a = jnp.concatenate([agg_ref[0:N, :], agg_ref[NP:NP + N, :]], axis=1)
    t = jnp.concatenate([table_ref[0:N, :], table_ref[NP:NP + N, :]], axis=1)
    conv_ref[...] = dinv[:, None] * (a + t) + b_ref[...]


def _bn_relu(cv, g_ref, be_ref):
    m = jnp.mean(cv, axis=0)
    v = jnp.mean((cv - m[None, :]) ** 2, axis=0)
    hn = (cv - m[None, :]) * lax.rsqrt(v + EPS)[None, :] * g_ref[...] + be_ref[...]
    return jnp.maximum(hn, 0.0)


def _bn_mm_body(conv_ref, g_ref, be_ref, w_ref, deg_ref, table_ref):
    h = _bn_relu(conv_ref[...], g_ref, be_ref)
    h2 = jnp.dot(h, w_ref[...], preferred_element_type=jnp.float32)
    dinv = _dinv_from_deg(deg_ref[...])
    _write_table(table_ref, h2 * dinv[:, None])


def _bn_final_body(conv_ref, g_ref, be_ref, o_ref):
    o_ref[...] = _bn_relu(conv_ref[...], g_ref, be_ref)


def _tc(body, out_shape):
    return pl.pallas_call(body, out_shape=out_shape)


def kernel(x, edge_index, W1, b1, g1, be1, W2, b2, g2, be2):
    f32 = jnp.float32
    src = edge_index[0]
    dst = edge_index[1]
    pad = jnp.full((EP - E,), N, jnp.int32)
    srcp = jnp.concatenate([src, pad])
    dstp = jnp.concatenate([dst, pad])
    isrc = jnp.concatenate([srcp, srcp + NP]).reshape(2 * NCHUNK, 128)
    idst = dstp.reshape(NCHUNK, 128)
    b1r, g1r, be1r = b1.reshape(1, HID), g1.reshape(1, HID), be1.reshape(1, HID)
    b2r, g2r, be2r = b2.reshape(1, HID), g2.reshape(1, HID), be2.reshape(1, HID)

    deg_parts = _deg_kernel(idst)
    h1 = _tc(_mm_body, jax.ShapeDtypeStruct((N, HID), f32))(x, W1)
    table1 = _tc(_prep_body, jax.ShapeDtypeStruct((2 * NP, 128), f32))(
        h1, deg_parts)
    agg1 = _agg_kernel(table1, isrc, idst)
    conv1 = _tc(_conv_body, jax.ShapeDtypeStruct((N, HID), f32))(
        agg1, table1, deg_parts, b1r)
    table2 = _tc(_bn_mm_body, jax.ShapeDtypeStruct((2 * NP, 128), f32))(
        conv1, g1r, be1r, W2, deg_parts)
    agg2 = _agg_kernel(table2, isrc, idst)
    conv2 = _tc(_conv_body, jax.ShapeDtypeStruct((N, HID), f32))(
        agg2, table2, deg_parts, b2r)
    out = _tc(_bn_final_body, jax.ShapeDtypeStruct((N, HID), f32))(
        conv2, g2r, be2r)
    return out


# speedup vs baseline: 8.6238x; 1.0135x over previous
import functools

import jax
import jax.numpy as jnp
from jax import lax
from jax.experimental import pallas as pl
from jax.experimental.pallas import tpu as pltpu
from jax.experimental.pallas import tpu_sc as plsc

N = 10000
NP = 10240
E = 320000
EP = 327680
NCHUNK = EP // 128
CPW_AGG = NCHUNK // 16
CPW_DEG = NCHUNK // 32
RPS = NP // 16
IN_DIM = 128
HID = 256
EPS = 1e-5

_MESH = plsc.VectorSubcoreMesh(core_axis_name="c", subcore_axis_name="s",
                               num_cores=2, num_subcores=16)


@functools.partial(
    pl.kernel,
    out_type=jax.ShapeDtypeStruct((2 * NP, 128), jnp.float32),
    mesh=_MESH,
    scratch_types=[
        pltpu.VMEM((CPW_DEG, 128), jnp.int32),
        pltpu.VMEM((128, 128), jnp.float32),
        pltpu.VMEM_SHARED((NP, 128), jnp.float32),
    ],
)
def _deg_kernel(idst_hbm, out_hbm, idx_v, ones_v, acc):
    c = lax.axis_index("c")
    s = lax.axis_index("s")

    @pl.loop(0, 128)
    def _(i):
        for j in range(8):
            ones_v[i, pl.ds(j * 16, 16)] = jnp.zeros((16,), jnp.float32)

    for k in range(RPS // 128):
        pltpu.sync_copy(ones_v, acc.at[pl.ds(s * RPS + k * 128, 128)])

    w = s * 2 + c
    pltpu.sync_copy(idst_hbm.at[pl.ds(w * CPW_DEG, CPW_DEG)], idx_v)
    plsc.subcore_barrier()

    @pl.loop(0, 128)
    def _(i):
        for j in range(8):
            ones_v[i, pl.ds(j * 16, 16)] = jnp.full((16,), 1.0, jnp.float32)

    @pl.loop(0, CPW_DEG)
    def _(j):
        pltpu.sync_copy(ones_v, acc.at[idx_v.at[j]], add=True)

    plsc.subcore_barrier()
    pltpu.sync_copy(acc.at[pl.ds(s * RPS, RPS)],
                    out_hbm.at[pl.ds(c * NP + s * RPS, RPS)])


IB = 16
NB = CPW_AGG // IB


@functools.partial(
    pl.kernel,
    out_type=jax.ShapeDtypeStruct((2 * NP, 128), jnp.float32),
    mesh=_MESH,
    scratch_types=[
        pltpu.VMEM((IB, 128), jnp.int32),
        pltpu.VMEM((IB, 128), jnp.int32),
        pltpu.VMEM((IB, 128), jnp.int32),
        pltpu.VMEM((IB, 128), jnp.int32),
        pltpu.VMEM((128, 128), jnp.float32),
        pltpu.VMEM((128, 128), jnp.float32),
        pltpu.VMEM_SHARED((NP, 128), jnp.float32),
        pltpu.SemaphoreType.DMA,
        pltpu.SemaphoreType.DMA,
        pltpu.SemaphoreType.DMA,
        pltpu.SemaphoreType.DMA,
    ],
)
def _agg_kernel(table_hbm, isrc_hbm, idst_hbm, out_hbm,
                isrc_v0, idst_v0, isrc_v1, idst_v1, rows_a, rows_b, acc,
                sem_a, sem_b, sem_ia, sem_ib):
    c = lax.axis_index("c")
    s = lax.axis_index("s")

    @pl.loop(0, 128)
    def _(i):
        for j in range(8):
            rows_a[i, pl.ds(j * 16, 16)] = jnp.zeros((16,), jnp.float32)

    for k in range(RPS // 128):
        pltpu.sync_copy(rows_a, acc.at[pl.ds(s * RPS + k * 128, 128)])

    plsc.subcore_barrier()

    base0 = s * CPW_AGG
    pltpu.async_copy(isrc_hbm.at[pl.ds(c * NCHUNK + base0, IB)], isrc_v0,
                     sem_ia)
    pltpu.async_copy(idst_hbm.at[pl.ds(base0, IB)], idst_v0, sem_ia)

    def _inner(isrc_v, idst_v):
        @pl.loop(0, IB, step=2)
        def _(j):
            cp_a = pltpu.async_copy(table_hbm.at[isrc_v.at[j]], rows_a, sem_a)
            cp_b = pltpu.async_copy(table_hbm.at[isrc_v.at[j + 1]], rows_b,
                                    sem_b)
            cp_a.wait()
            pltpu.sync_copy(rows_a, acc.at[idst_v.at[j]], add=True)
            cp_b.wait()
            pltpu.sync_copy(rows_b, acc.at[idst_v.at[j + 1]], add=True)

    @pl.loop(0, NB, step=2)
    def _(t):
        base_n = s * CPW_AGG + (t + 1) * IB
        pltpu.async_copy(isrc_hbm.at[pl.ds(c * NCHUNK + base_n, IB)],
                         isrc_v1, sem_ib)
        pltpu.async_copy(idst_hbm.at[pl.ds(base_n, IB)], idst_v1, sem_ib)
        pltpu.make_async_copy(idst_hbm.at[pl.ds(base_n, IB)], isrc_v0,
                              sem_ia).wait()
        pltpu.make_async_copy(idst_hbm.at[pl.ds(base_n, IB)], idst_v0,
                              sem_ia).wait()
        _inner(isrc_v0, idst_v0)

        @pl.when(t + 2 < NB)
        def _():
            base_2 = s * CPW_AGG + (t + 2) * IB
            pltpu.async_copy(isrc_hbm.at[pl.ds(c * NCHUNK + base_2, IB)],
                             isrc_v0, sem_ia)
            pltpu.async_copy(idst_hbm.at[pl.ds(base_2, IB)], idst_v0, sem_ia)

        pltpu.make_async_copy(idst_hbm.at[pl.ds(base_n, IB)], isrc_v1,
                              sem_ib).wait()
        pltpu.make_async_copy(idst_hbm.at[pl.ds(base_n, IB)], idst_v1,
                              sem_ib).wait()
        _inner(isrc_v1, idst_v1)

    plsc.subcore_barrier()
    pltpu.sync_copy(acc.at[pl.ds(s * RPS, RPS)],
                    out_hbm.at[pl.ds(c * NP + s * RPS, RPS)])


def _dinv_from_deg(deg_parts):
    deg = jnp.max(deg_parts[0:NP] + deg_parts[NP:2 * NP], axis=1) + 1.0
    return lax.rsqrt(deg)[:N]


def _write_table(table_ref, hp):
    table_ref[0:N, :] = hp[:, :128]
    table_ref[N:NP, :] = jnp.zeros((NP - N, 128), jnp.float32)
    table_ref[NP:NP + N, :] = hp[:, 128:]
    table_ref[NP + N:2 * NP, :] = jnp.zeros((NP - N, 128), jnp.float32)


def _mm_body(x_ref, w_ref, o_ref):
    o_ref[...] = jnp.dot(x_ref[...], w_ref[...],
                         preferred_element_type=jnp.float32)


def _prep_body(h_ref, deg_ref, table_ref):
    dinv = _dinv_from_deg(deg_ref[...])
    _write_table(table_ref, h_ref[...] * dinv[:, None])


def _conv_body(agg_ref, table_ref, deg_ref, b_ref, conv_ref):
    dinv = _dinv_from_deg(deg_ref[...])
    a = jnp.concatenate([agg_ref[0:N, :], agg_ref[NP:NP + N, :]], axis=1)
    t = jnp.concatenate([table_ref[0:N, :], table_ref[NP:NP + N, :]], axis=1)
    conv_ref[...] = dinv[:, None] * (a + t) + b_ref[...]


def _bn_relu(cv, g_ref, be_ref):
    m = jnp.mean(cv, axis=0)
    v = jnp.mean((cv - m[None, :]) ** 2, axis=0)
    hn = (cv - m[None, :]) * lax.rsqrt(v + EPS)[None, :] * g_ref[...] + be_ref[...]
    return jnp.maximum(hn, 0.0)


def _bn_mm_body(conv_ref, g_ref, be_ref, w_ref, deg_ref, table_ref):
    h = _bn_relu(conv_ref[...], g_ref, be_ref)
    h2 = jnp.dot(h, w_ref[...], preferred_element_type=jnp.float32)
    dinv = _dinv_from_deg(deg_ref[...])
    _write_table(table_ref, h2 * dinv[:, None])


def _bn_final_body(conv_ref, g_ref, be_ref, o_ref):
    o_ref[...] = _bn_relu(conv_ref[...], g_ref, be_ref)


def _tc(body, out_shape):
    return pl.pallas_call(body, out_shape=out_shape)


def kernel(x, edge_index, W1, b1, g1, be1, W2, b2, g2, be2):
    f32 = jnp.float32
    src = edge_index[0]
    dst = edge_index[1]
    pad = jnp.full((EP - E,), N, jnp.int32)
    srcp = jnp.concatenate([src, pad])
    dstp = jnp.concatenate([dst, pad])
    isrc = jnp.concatenate([srcp, srcp + NP]).reshape(2 * NCHUNK, 128)
    idst = dstp.reshape(NCHUNK, 128)
    b1r, g1r, be1r = b1.reshape(1, HID), g1.reshape(1, HID), be1.reshape(1, HID)
    b2r, g2r, be2r = b2.reshape(1, HID), g2.reshape(1, HID), be2.reshape(1, HID)

    deg_parts = _deg_kernel(idst)
    h1 = _tc(_mm_body, jax.ShapeDtypeStruct((N, HID), f32))(x, W1)
    table1 = _tc(_prep_body, jax.ShapeDtypeStruct((2 * NP, 128), f32))(
        h1, deg_parts)
    agg1 = _agg_kernel(table1, isrc, idst)
    conv1 = _tc(_conv_body, jax.ShapeDtypeStruct((N, HID), f32))(
        agg1, table1, deg_parts, b1r)
    table2 = _tc(_bn_mm_body, jax.ShapeDtypeStruct((2 * NP, 128), f32))(
        conv1, g1r, be1r, W2, deg_parts)
    agg2 = _agg_kernel(table2, isrc, idst)
    conv2 = _tc(_conv_body, jax.ShapeDtypeStruct((N, HID), f32))(
        agg2, table2, deg_parts, b2r)
    out = _tc(_bn_final_body, jax.ShapeDtypeStruct((N, HID), f32))(
        conv2, g2r, be2r)
    return out
